# baseline (speedup 1.0000x reference)
import functools

import jax
import jax.numpy as jnp
from jax import lax
from jax.experimental import pallas as pl
from jax.experimental.pallas import tpu as pltpu
from jax.experimental.pallas import tpu_sc as plsc

SIGMA = 0.5
N = 10000
E = 320000
D_IN = 128
H = 64
D_EDGE = 16
C = 40

NC = 2
NS = 16
NW = NC * NS
L = 16

EPT = E // NW
BLK = 80
NBLK = EPT // BLK
NG = BLK // L
NBUF = 5
NOUT = NBLK // NBUF

N2 = 10016
ROW = 80
NEG = -3.0e38

_mesh = plsc.VectorSubcoreMesh(core_axis_name="c", subcore_axis_name="s")
_SC_PARAMS = pltpu.CompilerParams(
    use_tc_tiling_on_sc=False, needs_layout_passes=False)


def _tc_in_body(x_ref, w_ref, b_ref, o_ref):
    o_ref[...] = (
        jnp.dot(x_ref[...], w_ref[...], preferred_element_type=jnp.float32)
        + b_ref[...]
    )


def _tc_ec_body(ea_ref, we1_ref, mu1_ref, we2_ref, mu2_ref, o1_ref, o2_ref):
    ea = ea_ref[...]
    o1_ref[...] = jnp.dot(ea, we1_ref[...], preferred_element_type=jnp.float32) - mu1_ref[...]
    o2_ref[...] = jnp.dot(ea, we2_ref[...], preferred_element_type=jnp.float32) - mu2_ref[...]


def _tc_cm_body(mp_ref, o_ref):
    o_ref[...] = jnp.max(mp_ref[...], axis=0, keepdims=True)


def _tc_out_body(p_ref, h_ref, wo_ref, bo_ref, o_ref):
    acc = p_ref[0] + p_ref[1]
    num = acc[:N, :H]
    den = acc[:N, H:H + 1]
    agg = (2.0 * num) / (den + 1e-12)
    out = jnp.dot(agg, wo_ref[...], preferred_element_type=jnp.float32)
    o_ref[...] = jnp.maximum(out + bo_ref[...], 0.0) + h_ref[...]


def _tc_out_cls_body(p_ref, h_ref, wo_ref, bo_ref, wc_ref, bc_ref, o_ref):
    acc = p_ref[0] + p_ref[1]
    num = acc[:N, :H]
    den = acc[:N, H:H + 1]
    agg = (2.0 * num) / (den + 1e-12)
    out = jnp.dot(agg, wo_ref[...], preferred_element_type=jnp.float32)
    h2 = jnp.maximum(out + bo_ref[...], 0.0) + h_ref[...]
    o_ref[...] = (
        jnp.dot(h2, wc_ref[...], preferred_element_type=jnp.float32)
        + bc_ref[...]
    )


def _unpack_ec(ecbuf, row):
    return tuple(ecbuf[row, pl.ds(q * L, L)] for q in range(H // L))


def _tc_call(body, out_shape, *args):
    return pl.pallas_call(
        body, out_shape=jax.ShapeDtypeStruct(out_shape, jnp.float32)
    )(*args)


def _sc_p1_body(h_hbm, ec_hbm, src_hbm, dst_hbm,
                lg_hbm, mp_hbm,
                srcall, dstall, lall, mtab, hset, ecset, gsem, esem):
    ci = lax.axis_index("c")
    si = lax.axis_index("s")
    wid = si * NC + ci
    tbase = wid * EPT
    lanes = lax.broadcasted_iota(jnp.int32, (L,), 0)

    @pl.loop(0, N2 // L)
    def _init(i):
        mtab[pl.ds(i * L, L)] = jnp.full((L,), NEG, jnp.float32)

    pltpu.sync_copy(src_hbm.at[pl.ds(tbase, EPT)], srcall)
    pltpu.sync_copy(dst_hbm.at[pl.ds(tbase, EPT)], dstall)

    def _issue(b, s):
        pltpu.async_copy(
            h_hbm.at[srcall.at[pl.ds(b * BLK, BLK)]], hset.at[s],
            gsem.at[s])
        pltpu.async_copy(
            ec_hbm.at[pl.ds(tbase + b * BLK, BLK)], ecset.at[s],
            esem.at[s])

    def _wait(b, s):
        pltpu.make_async_copy(
            h_hbm.at[srcall.at[pl.ds(b * BLK, BLK)]], hset.at[s],
            gsem.at[s]).wait()
        pltpu.make_async_copy(
            ec_hbm.at[pl.ds(tbase + b * BLK, BLK)], ecset.at[s],
            esem.at[s]).wait()

    for s in range(NBUF):
        _issue(s, s)

    @pl.loop(0, NOUT)
    def _outer(o):
        for s in range(NBUF):
            b = o * NBUF + s
            _wait(b, s)
            hbuf = hset.at[s]
            ecbuf = ecset.at[s]

            @pl.loop(0, NG)
            def _grp(g):
                off = g * L
                lacc = jnp.zeros((L,), jnp.float32)
                for e in range(L):
                    row = off + e
                    ecq = _unpack_ec(ecbuf, row)
                    sq = jnp.zeros((L,), jnp.float32)
                    for q in range(H // L):
                        d = hbuf[row, pl.ds(q * L, L)] + ecq[q]
                        sq = sq + d * d
                    logit = -2.0 * jnp.sum(sq)
                    lacc = jnp.where(lanes == e, logit, lacc)
                lall[pl.ds(b * BLK + off, L)] = lacc
                dstv = dstall[pl.ds(b * BLK + off, L)]

                def _cond(pend):
                    return jnp.any(pend)

                def _body(pend):
                    cur = plsc.load_gather(mtab, [dstv])
                    write = jnp.logical_and(pend, lacc > cur)
                    plsc.store_scatter(mtab, [dstv], lacc, mask=write)
                    cur2 = plsc.load_gather(mtab, [dstv])
                    return lacc > cur2

                lax.while_loop(_cond, _body, jnp.ones((L,), jnp.bool_))

            @pl.when(o < NOUT - 1)
            def _next():
                _issue(b + NBUF, s)

    pltpu.sync_copy(lall, lg_hbm.at[pl.ds(tbase, EPT)])
    pltpu.sync_copy(mtab, mp_hbm.at[wid])


def _sc_p2_body(h_hbm, ec_hbm, src_hbm, dst_hbm, lg_hbm, m_hbm, z_hbm,
                p_hbm,
                srcset, mtab, hset, ecset, dstset, lgset, dsts, whb,
                acc, gsem, esem, ssem, xsem):
    ci = lax.axis_index("c")
    si = lax.axis_index("s")
    wid = si * NC + ci
    tbase = wid * EPT
    lanes = lax.broadcasted_iota(jnp.int32, (L,), 0)

    rows_per_tile = N2 // NS
    pltpu.sync_copy(z_hbm.at[pl.ds(si * rows_per_tile, rows_per_tile)],
                    acc.at[pl.ds(si * rows_per_tile, rows_per_tile)])
    pltpu.sync_copy(m_hbm.at[0], mtab)
    plsc.subcore_barrier()

    def _issue_src(b, s):
        pltpu.async_copy(
            src_hbm.at[pl.ds(tbase + b * BLK, BLK)], srcset.at[s],
            xsem.at[s])

    def _wait_src(b, s):
        pltpu.make_async_copy(
            src_hbm.at[pl.ds(tbase + b * BLK, BLK)], srcset.at[s],
            xsem.at[s]).wait()

    def _issue_lin(b, s):
        pltpu.async_copy(
            ec_hbm.at[pl.ds(tbase + b * BLK, BLK)], ecset.at[s],
            esem.at[s])
        pltpu.async_copy(
            dst_hbm.at[pl.ds(tbase + b * BLK, BLK)], dstset.at[s],
            esem.at[s])
        pltpu.async_copy(
            lg_hbm.at[pl.ds(tbase + b * BLK, BLK)], lgset.at[s],
            esem.at[s])

    def _wait_lin(b, s):
        pltpu.make_async_copy(
            ec_hbm.at[pl.ds(tbase + b * BLK, BLK)], ecset.at[s],
            esem.at[s]).wait()
        pltpu.make_async_copy(
            dst_hbm.at[pl.ds(tbase + b * BLK, BLK)], dstset.at[s],
            esem.at[s]).wait()
        pltpu.make_async_copy(
            lg_hbm.at[pl.ds(tbase + b * BLK, BLK)], lgset.at[s],
            esem.at[s]).wait()

    def _gissue(s):
        pltpu.async_copy(h_hbm.at[srcset.at[s]], hset.at[s], gsem.at[s])

    def _gwait(s):
        pltpu.make_async_copy(
            h_hbm.at[srcset.at[s]], hset.at[s], gsem.at[s]).wait()

    def _sdrain(p):
        pltpu.make_async_copy(
            whb.at[p], acc.at[dsts.at[p]], ssem.at[p]).wait()

    for s in range(NBUF):
        _issue_src(s, s)
        _issue_lin(s, s)
    for s in range(3):
        _wait_src(s, s)
        _gissue(s)

    @pl.loop(0, NOUT)
    def _outer(o):
        for s in range(NBUF):
            b = o * NBUF + s
            p = s % 2
            s3 = (s + 3) % NBUF
            _wait_lin(b, s)
            _gwait(s)

            if s >= 2:
                _sdrain(p)
            else:
                @pl.when(o > 0)
                def _drain():
                    _sdrain(p)

            hbuf = hset.at[s]
            ecbuf = ecset.at[s]

            @pl.loop(0, NG)
            def _grp(g):
                off = g * L
                dstv = dstset[s, pl.ds(off, L)]
                mv = plsc.load_gather(mtab, [dstv])
                wv = jnp.exp(lgset[s, pl.ds(off, L)] - mv)
                dsts[p, pl.ds(off, L)] = dstv
                for e in range(L):
                    row = off + e
                    ecq = _unpack_ec(ecbuf, row)
                    ws = wv[e]
                    for q in range(H // L):
                        d = hbuf[row, pl.ds(q * L, L)] + ecq[q]
                        whb[p, row, pl.ds(q * L, L)] = ws * d
                    whb[p, row, pl.ds(H, L)] = jnp.where(
                        lanes == 0, ws, 0.0)

            pltpu.async_copy(whb.at[p], acc.at[dsts.at[p]], ssem.at[p],
                             add=True)

            @pl.when(o < NOUT - 1)
            def _next():
                _issue_src(b + NBUF, s)
                _issue_lin(b + NBUF, s)

            if s < NBUF - 3:
                _wait_src(b + 3, s3)
                _gissue(s3)
            else:
                @pl.when(o < NOUT - 1)
                def _g3():
                    _wait_src(b + 3, s3)
                    _gissue(s3)

    for p in range(2):
        _sdrain(p)

    plsc.subcore_barrier()

    @pl.when(si == 0)
    def _dump():
        pltpu.sync_copy(acc, p_hbm.at[ci])


_sc_p1 = functools.partial(
    pl.kernel,
    _sc_p1_body,
    out_type=(
        jax.ShapeDtypeStruct((E,), jnp.float32),
        jax.ShapeDtypeStruct((NW, N2), jnp.float32),
    ),
    mesh=_mesh,
    compiler_params=_SC_PARAMS,
    scratch_types=[
        pltpu.VMEM((EPT,), jnp.int32),
        pltpu.VMEM((EPT,), jnp.int32),
        pltpu.VMEM((EPT,), jnp.float32),
        pltpu.VMEM((N2,), jnp.float32),
        pltpu.VMEM((NBUF, BLK, H), jnp.float32),
        pltpu.VMEM((NBUF, BLK, H), jnp.float32),
        pltpu.SemaphoreType.DMA((NBUF,)),
        pltpu.SemaphoreType.DMA((NBUF,)),
    ],
)()

_sc_p2 = functools.partial(
    pl.kernel,
    _sc_p2_body,
    out_type=jax.ShapeDtypeStruct((NC, N2, ROW), jnp.float32),
    mesh=_mesh,
    compiler_params=_SC_PARAMS,
    scratch_types=[
        pltpu.VMEM((NBUF, BLK), jnp.int32),
        pltpu.VMEM((N2,), jnp.float32),
        pltpu.VMEM((NBUF, BLK, H), jnp.float32),
        pltpu.VMEM((NBUF, BLK, H), jnp.float32),
        pltpu.VMEM((NBUF, BLK), jnp.int32),
        pltpu.VMEM((NBUF, BLK), jnp.float32),
        pltpu.VMEM((2, BLK), jnp.int32),
        pltpu.VMEM((2, BLK, ROW), jnp.float32),
        pltpu.VMEM_SHARED((N2, ROW), jnp.float32),
        pltpu.SemaphoreType.DMA((NBUF,)),
        pltpu.SemaphoreType.DMA((NBUF,)),
        pltpu.SemaphoreType.DMA((2,)),
        pltpu.SemaphoreType.DMA((NBUF,)),
    ],
)()


def kernel(x, edge_index, edge_attr, W_in, b_in, mu1, We1, Wo1, bo1,
           mu2, We2, Wo2, bo2, W_cls, b_cls):
    src = edge_index[0]
    dst = edge_index[1]
    zeros = jnp.zeros((N2, ROW), jnp.float32)

    ec1, ec2 = pl.pallas_call(
        _tc_ec_body,
        out_shape=(jax.ShapeDtypeStruct((E, H), jnp.float32),
                   jax.ShapeDtypeStruct((E, H), jnp.float32)),
        grid=(40,),
        in_specs=[
            pl.BlockSpec((E // 40, D_EDGE), lambda i: (i, 0)),
            pl.BlockSpec((D_EDGE, H), lambda i: (0, 0)),
            pl.BlockSpec((1, H), lambda i: (0, 0)),
            pl.BlockSpec((D_EDGE, H), lambda i: (0, 0)),
            pl.BlockSpec((1, H), lambda i: (0, 0)),
        ],
        out_specs=(pl.BlockSpec((E // 40, H), lambda i: (i, 0)),
                   pl.BlockSpec((E // 40, H), lambda i: (i, 0))),
    )(edge_attr, We1, mu1, We2, mu2)

    h = _tc_call(_tc_in_body, (N, H), x, W_in, b_in.reshape(1, H))

    lg, mp = _sc_p1(h, ec1, src, dst)
    m = _tc_call(_tc_cm_body, (1, N2), mp)
    p = _sc_p2(h, ec1, src, dst, lg, m, zeros)
    h = _tc_call(_tc_out_body, (N, H), p, h, Wo1, bo1.reshape(1, H))

    lg, mp = _sc_p1(h, ec2, src, dst)
    m = _tc_call(_tc_cm_body, (1, N2), mp)
    p = _sc_p2(h, ec2, src, dst, lg, m, zeros)
    return _tc_call(_tc_out_cls_body, (N, C), p, h, Wo2,
                    bo2.reshape(1, H), W_cls, b_cls.reshape(1, C))


# speedup vs baseline: 1.1643x; 1.1643x over previous
import functools

import jax
import jax.numpy as jnp
from jax import lax
from jax.experimental import pallas as pl
from jax.experimental.pallas import tpu as pltpu
from jax.experimental.pallas import tpu_sc as plsc

SIGMA = 0.5
N = 10000
E = 320000
D_IN = 128
H = 64
D_EDGE = 16
C = 40

NC = 2
NS = 16
NW = NC * NS
L = 16

EPT = E // NW
BLK = 80
NBLK = EPT // BLK
NG = BLK // L
NBUF = 5
NOUT = NBLK // NBUF

N2 = 10016
ROW = 80
NEG = -3.0e38

_mesh = plsc.VectorSubcoreMesh(core_axis_name="c", subcore_axis_name="s")
_SC_PARAMS = pltpu.CompilerParams(
    use_tc_tiling_on_sc=False, needs_layout_passes=False)


def _tc_in_body(x_ref, w_ref, b_ref, o_ref):
    o_ref[...] = (
        jnp.dot(x_ref[...], w_ref[...], preferred_element_type=jnp.float32)
        + b_ref[...]
    )


def _tc_ec_body(ea_ref, we1_ref, mu1_ref, we2_ref, mu2_ref, o1_ref, o2_ref):
    ea = ea_ref[...]
    o1_ref[...] = jnp.dot(ea, we1_ref[...], preferred_element_type=jnp.float32) - mu1_ref[...]
    o2_ref[...] = jnp.dot(ea, we2_ref[...], preferred_element_type=jnp.float32) - mu2_ref[...]


def _tc_cm_body(mp_ref, o_ref):
    o_ref[...] = jnp.max(mp_ref[...], axis=0, keepdims=True)


def _tc_out_body(p_ref, h_ref, wo_ref, bo_ref, o_ref):
    acc = p_ref[0] + p_ref[1]
    num = acc[:N, :H]
    den = acc[:N, H:H + 1]
    agg = (2.0 * num) / (den + 1e-12)
    out = jnp.dot(agg, wo_ref[...], preferred_element_type=jnp.float32)
    o_ref[...] = jnp.maximum(out + bo_ref[...], 0.0) + h_ref[...]


def _tc_out_cls_body(p_ref, h_ref, wo_ref, bo_ref, wc_ref, bc_ref, o_ref):
    acc = p_ref[0] + p_ref[1]
    num = acc[:N, :H]
    den = acc[:N, H:H + 1]
    agg = (2.0 * num) / (den + 1e-12)
    out = jnp.dot(agg, wo_ref[...], preferred_element_type=jnp.float32)
    h2 = jnp.maximum(out + bo_ref[...], 0.0) + h_ref[...]
    o_ref[...] = (
        jnp.dot(h2, wc_ref[...], preferred_element_type=jnp.float32)
        + bc_ref[...]
    )


def _unpack_ec(ecbuf, row):
    return tuple(ecbuf[row, pl.ds(q * L, L)] for q in range(H // L))


def _tc_call(body, out_shape, *args):
    return pl.pallas_call(
        body, out_shape=jax.ShapeDtypeStruct(out_shape, jnp.float32)
    )(*args)


def _sc_p1_body(h_hbm, ec_hbm, src_hbm, dst_hbm,
                lg_hbm, mp_hbm, df_hbm,
                srcall, dstall, lall, mtab, hset, ecset, dbuf,
                gsem, esem, wsem):
    ci = lax.axis_index("c")
    si = lax.axis_index("s")
    wid = si * NC + ci
    tbase = wid * EPT
    lanes = lax.broadcasted_iota(jnp.int32, (L,), 0)

    @pl.loop(0, N2 // L)
    def _init(i):
        mtab[pl.ds(i * L, L)] = jnp.full((L,), NEG, jnp.float32)

    pltpu.sync_copy(src_hbm.at[pl.ds(tbase, EPT)], srcall)
    pltpu.sync_copy(dst_hbm.at[pl.ds(tbase, EPT)], dstall)

    def _issue(b, s):
        pltpu.async_copy(
            h_hbm.at[srcall.at[pl.ds(b * BLK, BLK)]], hset.at[s],
            gsem.at[s])
        pltpu.async_copy(
            ec_hbm.at[pl.ds(tbase + b * BLK, BLK)], ecset.at[s],
            esem.at[s])

    def _wait(b, s):
        pltpu.make_async_copy(
            h_hbm.at[srcall.at[pl.ds(b * BLK, BLK)]], hset.at[s],
            gsem.at[s]).wait()
        pltpu.make_async_copy(
            ec_hbm.at[pl.ds(tbase + b * BLK, BLK)], ecset.at[s],
            esem.at[s]).wait()

    for s in range(NBUF):
        _issue(s, s)

    @pl.loop(0, NOUT)
    def _outer(o):
        for s in range(NBUF):
            b = o * NBUF + s
            _wait(b, s)

            @pl.when(o > 0)
            def _wdrain():
                pltpu.make_async_copy(
                    dbuf.at[s],
                    df_hbm.at[pl.ds(tbase + (b - NBUF) * BLK, BLK)],
                    wsem.at[s]).wait()

            hbuf = hset.at[s]
            ecbuf = ecset.at[s]

            @pl.loop(0, NG)
            def _grp(g):
                off = g * L
                lacc = jnp.zeros((L,), jnp.float32)
                for e in range(L):
                    row = off + e
                    ecq = _unpack_ec(ecbuf, row)
                    sq = jnp.zeros((L,), jnp.float32)
                    ds_ = []
                    for q in range(H // L):
                        d = hbuf[row, pl.ds(q * L, L)] + ecq[q]
                        ds_.append(d)
                        sq = sq + d * d
                    dbuf[s, row, pl.ds(0, L)] = plsc.bitcast(
                        plsc.pack(ds_[0], ds_[1],
                                  format=plsc.PackFormat.INTERLEAVED),
                        jnp.int32)
                    dbuf[s, row, pl.ds(L, L)] = plsc.bitcast(
                        plsc.pack(ds_[2], ds_[3],
                                  format=plsc.PackFormat.INTERLEAVED),
                        jnp.int32)
                    logit = -2.0 * jnp.sum(sq)
                    lacc = jnp.where(lanes == e, logit, lacc)
                lall[pl.ds(b * BLK + off, L)] = lacc
                dstv = dstall[pl.ds(b * BLK + off, L)]

                def _cond(pend):
                    return jnp.any(pend)

                def _body(pend):
                    cur = plsc.load_gather(mtab, [dstv])
                    write = jnp.logical_and(pend, lacc > cur)
                    plsc.store_scatter(mtab, [dstv], lacc, mask=write)
                    cur2 = plsc.load_gather(mtab, [dstv])
                    return lacc > cur2

                lax.while_loop(_cond, _body, jnp.ones((L,), jnp.bool_))

            pltpu.async_copy(
                dbuf.at[s], df_hbm.at[pl.ds(tbase + b * BLK, BLK)],
                wsem.at[s])

            @pl.when(o < NOUT - 1)
            def _next():
                _issue(b + NBUF, s)

    for s in range(NBUF):
        pltpu.make_async_copy(
            dbuf.at[s],
            df_hbm.at[pl.ds(tbase + ((NOUT - 1) * NBUF + s) * BLK, BLK)],
            wsem.at[s]).wait()
    pltpu.sync_copy(lall, lg_hbm.at[pl.ds(tbase, EPT)])
    pltpu.sync_copy(mtab, mp_hbm.at[wid])


def _sc_p2_body(df_hbm, dst_hbm, lg_hbm, m_hbm, z_hbm,
                p_hbm,
                mtab, dset, dstset, lgset, dsts, whb,
                acc, esem, ssem):
    ci = lax.axis_index("c")
    si = lax.axis_index("s")
    wid = si * NC + ci
    tbase = wid * EPT
    lanes = lax.broadcasted_iota(jnp.int32, (L,), 0)

    rows_per_tile = N2 // NS
    pltpu.sync_copy(z_hbm.at[pl.ds(si * rows_per_tile, rows_per_tile)],
                    acc.at[pl.ds(si * rows_per_tile, rows_per_tile)])
    pltpu.sync_copy(m_hbm.at[0], mtab)
    plsc.subcore_barrier()

    def _issue_lin(b, s):
        pltpu.async_copy(
            df_hbm.at[pl.ds(tbase + b * BLK, BLK)], dset.at[s],
            esem.at[s])
        pltpu.async_copy(
            dst_hbm.at[pl.ds(tbase + b * BLK, BLK)], dstset.at[s],
            esem.at[s])
        pltpu.async_copy(
            lg_hbm.at[pl.ds(tbase + b * BLK, BLK)], lgset.at[s],
            esem.at[s])

    def _wait_lin(b, s):
        pltpu.make_async_copy(
            df_hbm.at[pl.ds(tbase + b * BLK, BLK)], dset.at[s],
            esem.at[s]).wait()
        pltpu.make_async_copy(
            dst_hbm.at[pl.ds(tbase + b * BLK, BLK)], dstset.at[s],
            esem.at[s]).wait()
        pltpu.make_async_copy(
            lg_hbm.at[pl.ds(tbase + b * BLK, BLK)], lgset.at[s],
            esem.at[s]).wait()

    def _sdrain(s):
        pltpu.make_async_copy(
            whb.at[s], acc.at[dsts.at[s]], ssem.at[s]).wait()

    for s in range(NBUF):
        _issue_lin(s, s)

    @pl.loop(0, NOUT)
    def _outer(o):
        for s in range(NBUF):
            b = o * NBUF + s
            _wait_lin(b, s)

            @pl.when(o > 0)
            def _drain():
                _sdrain(s)

            dbuf = dset.at[s]

            @pl.loop(0, NG)
            def _grp(g):
                off = g * L
                dstv = dstset[s, pl.ds(off, L)]
                mv = plsc.load_gather(mtab, [dstv])
                wv = jnp.exp(lgset[s, pl.ds(off, L)] - mv)
                dsts[s, pl.ds(off, L)] = dstv
                for e in range(L):
                    row = off + e
                    ws = wv[e]
                    for hw in range(2):
                        dq = plsc.unpack(
                            plsc.bitcast(dbuf[row, pl.ds(hw * L, L)],
                                         jnp.bfloat16),
                            format=plsc.PackFormat.INTERLEAVED)
                        whb[s, row, pl.ds(2 * hw * L, L)] = ws * dq[0]
                        whb[s, row, pl.ds((2 * hw + 1) * L, L)] = ws * dq[1]
                    whb[s, row, pl.ds(H, L)] = jnp.where(
                        lanes == 0, ws, 0.0)

            pltpu.async_copy(whb.at[s], acc.at[dsts.at[s]], ssem.at[s],
                             add=True)

            @pl.when(o < NOUT - 1)
            def _next():
                _issue_lin(b + NBUF, s)

    for s in range(NBUF):
        _sdrain(s)

    plsc.subcore_barrier()

    @pl.when(si == 0)
    def _dump():
        pltpu.sync_copy(acc, p_hbm.at[ci])


_sc_p1 = functools.partial(
    pl.kernel,
    _sc_p1_body,
    out_type=(
        jax.ShapeDtypeStruct((E,), jnp.float32),
        jax.ShapeDtypeStruct((NW, N2), jnp.float32),
        jax.ShapeDtypeStruct((E, 2 * L), jnp.int32),
    ),
    mesh=_mesh,
    compiler_params=_SC_PARAMS,
    scratch_types=[
        pltpu.VMEM((EPT,), jnp.int32),
        pltpu.VMEM((EPT,), jnp.int32),
        pltpu.VMEM((EPT,), jnp.float32),
        pltpu.VMEM((N2,), jnp.float32),
        pltpu.VMEM((NBUF, BLK, H), jnp.float32),
        pltpu.VMEM((NBUF, BLK, H), jnp.float32),
        pltpu.VMEM((NBUF, BLK, 2 * L), jnp.int32),
        pltpu.SemaphoreType.DMA((NBUF,)),
        pltpu.SemaphoreType.DMA((NBUF,)),
        pltpu.SemaphoreType.DMA((NBUF,)),
    ],
)()

_sc_p2 = functools.partial(
    pl.kernel,
    _sc_p2_body,
    out_type=jax.ShapeDtypeStruct((NC, N2, ROW), jnp.float32),
    mesh=_mesh,
    compiler_params=_SC_PARAMS,
    scratch_types=[
        pltpu.VMEM((N2,), jnp.float32),
        pltpu.VMEM((NBUF, BLK, 2 * L), jnp.int32),
        pltpu.VMEM((NBUF, BLK), jnp.int32),
        pltpu.VMEM((NBUF, BLK), jnp.float32),
        pltpu.VMEM((NBUF, BLK), jnp.int32),
        pltpu.VMEM((NBUF, BLK, ROW), jnp.float32),
        pltpu.VMEM_SHARED((N2, ROW), jnp.float32),
        pltpu.SemaphoreType.DMA((NBUF,)),
        pltpu.SemaphoreType.DMA((NBUF,)),
    ],
)()


def kernel(x, edge_index, edge_attr, W_in, b_in, mu1, We1, Wo1, bo1,
           mu2, We2, Wo2, bo2, W_cls, b_cls):
    src = edge_index[0]
    dst = edge_index[1]
    zeros = jnp.zeros((N2, ROW), jnp.float32)

    ec1, ec2 = pl.pallas_call(
        _tc_ec_body,
        out_shape=(jax.ShapeDtypeStruct((E, H), jnp.float32),
                   jax.ShapeDtypeStruct((E, H), jnp.float32)),
        grid=(40,),
        in_specs=[
            pl.BlockSpec((E // 40, D_EDGE), lambda i: (i, 0)),
            pl.BlockSpec((D_EDGE, H), lambda i: (0, 0)),
            pl.BlockSpec((1, H), lambda i: (0, 0)),
            pl.BlockSpec((D_EDGE, H), lambda i: (0, 0)),
            pl.BlockSpec((1, H), lambda i: (0, 0)),
        ],
        out_specs=(pl.BlockSpec((E // 40, H), lambda i: (i, 0)),
                   pl.BlockSpec((E // 40, H), lambda i: (i, 0))),
    )(edge_attr, We1, mu1, We2, mu2)

    h = _tc_call(_tc_in_body, (N, H), x, W_in, b_in.reshape(1, H))

    lg, mp, df = _sc_p1(h, ec1, src, dst)
    m = _tc_call(_tc_cm_body, (1, N2), mp)
    p = _sc_p2(df, dst, lg, m, zeros)
    h = _tc_call(_tc_out_body, (N, H), p, h, Wo1, bo1.reshape(1, H))

    lg, mp, df = _sc_p1(h, ec2, src, dst)
    m = _tc_call(_tc_cm_body, (1, N2), mp)
    p = _sc_p2(df, dst, lg, m, zeros)
    return _tc_call(_tc_out_cls_body, (N, C), p, h, Wo2,
                    bo2.reshape(1, H), W_cls, b_cls.reshape(1, C))
